# baseline (speedup 1.0000x reference)
import functools

import jax
import jax.numpy as jnp
from jax import lax
from jax.experimental import pallas as pl
from jax.experimental.pallas import tpu as pltpu
from jax.experimental.pallas import tpu_sc as plsc

N, E, D, DE, H = 10000, 160000, 256, 16, 512
NC, NS = 2, 16
NW = NC * NS
GC = 200
NPW = 320
NPAD = NW * NPW
SCH = 1600
UNR = 4
GB = 64
BE = 1600
EH0, EH1 = 83200, 76800


def _mesh():
    return plsc.VectorSubcoreMesh(
        core_axis_name="c", subcore_axis_name="s", num_cores=NC, num_subcores=NS
    )


_SC_GATHER_PARAMS = pltpu.CompilerParams(
    needs_layout_passes=False, use_tc_tiling_on_sc=False)
_SC_SCATTER_PARAMS = pltpu.CompilerParams(
    needs_layout_passes=False, use_tc_tiling_on_sc=False)


def _gather_body(ecount, x_hbm, row_hbm, col_hbm, gi_hbm, gj_hbm,
                 ridx, cidx, xi_v, xj_v, sem1, sem2):
    epw = ecount // NW
    wid = lax.axis_index("s") * NC + lax.axis_index("c")
    base = wid * epw

    def chunk(k, carry):
        off = base + k * GC
        pltpu.sync_copy(row_hbm.at[pl.ds(off, GC)], ridx)
        pltpu.sync_copy(col_hbm.at[pl.ds(off, GC)], cidx)
        cp1 = pltpu.async_copy(x_hbm.at[ridx], xi_v, sem1)
        cp2 = pltpu.async_copy(x_hbm.at[cidx], xj_v, sem2)
        cp1.wait()
        cp2.wait()
        pltpu.sync_copy(xi_v, gi_hbm.at[pl.ds(off, GC)])
        pltpu.sync_copy(xj_v, gj_hbm.at[pl.ds(off, GC)])
        return carry

    lax.fori_loop(0, epw // GC, chunk, 0)


def _bcast_last(v, lane15):
    dnums = lax.GatherDimensionNumbers(
        offset_dims=(), collapsed_slice_dims=(0,), start_index_map=(0,))
    return lax.gather(v, lane15.reshape(16, 1), dnums, (1,),
                      mode=lax.GatherScatterMode.PROMISE_IN_BOUNDS)


def _scatter_body(ecount, msg_hbm, col_hbm, init_hbm, scat_hbm,
                  cols0, cols1, mid0, mid1, mdst0, mdst1, rows0, rows1, out_v,
                  cnts, sg0, sg1, sc0, sc1):
    nch = ecount // SCH
    wid = lax.axis_index("s") * NC + lax.axis_index("c")
    lo = wid * NPW
    hi = lo + NPW
    pltpu.sync_copy(init_hbm.at[pl.ds(lo, NPW)], out_v.at[pl.ds(0, NPW)])
    zid = jnp.zeros((16,), jnp.int32)
    cols = (cols0, cols1)
    mid = (mid0, mid1)
    mdst = (mdst0, mdst1)
    rows = (rows0, rows1)
    sg = (sg0, sg1)
    sc = (sc0, sc1)

    def iz(j, carry):
        mid0[pl.ds(16 * j, 16)] = zid
        mid1[pl.ds(16 * j, 16)] = zid
        return carry

    lax.fori_loop(0, (SCH + 16) // 16, iz, 0)
    iota = lax.iota(jnp.int32, 16)
    hivec = jnp.zeros((16,), jnp.int32) + hi

    pltpu.async_copy(col_hbm.at[pl.ds(0, SCH)], cols[0], sc[0])

    def scan_issue(k, par):
        cbase = k * SCH
        pltpu.make_async_copy(col_hbm.at[pl.ds(cbase, SCH)], cols[par],
                              sc[par]).wait()

        @pl.when(k + 1 < nch)
        def _():
            pltpu.async_copy(col_hbm.at[pl.ds(cbase + SCH, SCH)],
                             cols[1 - par], sc[1 - par])

        cv0 = cols[par]
        midp = mid[par]
        mdstp = mdst[par]

        lane15 = jnp.zeros((16,), jnp.int32) + 15

        def scan(j, cntv):
            for u in range(UNR):
                o = 16 * (UNR * j + u)
                cv = cv0[pl.ds(o, 16)]
                m = (cv >= lo) & (cv < hi)
                cs = plsc.cumsum(jnp.where(m, 1, 0))
                idx = cntv + cs - 1
                plsc.store_scatter(midp, [idx], cbase + o + iota, mask=m)
                plsc.store_scatter(mdstp, [idx], cv, mask=m)
                cntv = cntv + _bcast_last(cs, lane15)
            return cntv

        cntv = lax.fori_loop(0, SCH // (16 * UNR), scan,
                             jnp.zeros((16,), jnp.int32))
        cnt = cntv[0]
        mdstp[pl.ds(cnt, 16)] = hivec
        cnts[par] = cnt
        pltpu.async_copy(msg_hbm.at[midp.at[pl.ds(0, GB)]], rows[par], sg[par])

    def accum(rowsp, mdstp, base, nrows):
        def grp(g, carry):
            g16 = g * 16
            dstv = mdstp[pl.ds(base + g16, 16)]
            for t in range(16):
                r = dstv[t] - lo
                for v in range(16):
                    sl = pl.ds(16 * v, 16)
                    out_v[r, sl] = jnp.maximum(out_v[r, sl],
                                               rowsp[g16 + t, sl])
            return carry

        lax.fori_loop(0, (nrows + 15) // 16, grp, 0)

    def process(k, par):
        midp = mid[par]
        mdstp = mdst[par]
        rowsp = rows[par]
        cnt = cnts[par]
        pltpu.make_async_copy(msg_hbm.at[midp.at[pl.ds(0, GB)]], rowsp,
                              sg[par]).wait()
        accum(rowsp, mdstp, 0, jnp.minimum(cnt, GB))

        def bat(b, carry):
            bb = b * GB
            pltpu.async_copy(msg_hbm.at[midp.at[pl.ds(bb, GB)]], rowsp,
                             sg[par]).wait()
            accum(rowsp, mdstp, bb, jnp.minimum(cnt - bb, GB))
            return carry

        lax.fori_loop(1, (cnt + GB - 1) // GB, bat, 0)

    def pair(kk, carry):
        k0 = 2 * kk
        scan_issue(k0, 0)

        @pl.when(k0 > 0)
        def _():
            process(k0 - 1, 1)

        scan_issue(k0 + 1, 1)
        process(k0, 0)
        return carry

    lax.fori_loop(0, nch // 2, pair, 0)
    process(nch - 1, 1)
    pltpu.sync_copy(out_v.at[pl.ds(0, NPW)], scat_hbm.at[pl.ds(lo, NPW)])


def _mlp_body(xi_ref, xj_ref, ea_ref, w1a, w1b, w1c, w1d, b1_ref, w2, b2_ref,
              msg_ref):
    xi = xi_ref[...]
    xj = xj_ref[...]
    dirn = (2.3094 * jnp.tanh(xj.astype(jnp.float32) - xi.astype(jnp.float32))
            ).astype(jnp.bfloat16)
    ea = ea_ref[...].astype(jnp.bfloat16)

    def dot(a, b):
        return lax.dot_general(a, b, (((1,), (0,)), ((), ())),
                               preferred_element_type=jnp.float32)

    acc = dot(xi, w1a[...]) + dot(xj, w1b[...]) + dot(ea, w1c[...]) + dot(
        dirn, w1d[...])
    h = jnp.maximum(acc + b1_ref[...], 0.0).astype(jnp.bfloat16)
    msg_ref[...] = dot(h, w2[...]) + b2_ref[...]


def _tail(scat_ref, x_ref, o_ref):
    o_ref[...] = 4.0 * jnp.tanh(scat_ref[...] + 0.1 * x_ref[...])


def _gather_half(x_bf, row_h, col_h, ecount):
    return pl.kernel(
        functools.partial(_gather_body, ecount),
        out_type=[
            jax.ShapeDtypeStruct((ecount, D), jnp.bfloat16),
            jax.ShapeDtypeStruct((ecount, D), jnp.bfloat16),
        ],
        mesh=_mesh(),
        compiler_params=_SC_GATHER_PARAMS,
        cost_estimate=pl.CostEstimate(
            flops=0, transcendentals=0,
            bytes_accessed=ecount * (2 * D * 2 + 8) + N * D * 2),
        scratch_types=[
            pltpu.VMEM((GC,), jnp.int32),
            pltpu.VMEM((GC,), jnp.int32),
            pltpu.VMEM((GC, D), jnp.bfloat16),
            pltpu.VMEM((GC, D), jnp.bfloat16),
            pltpu.SemaphoreType.DMA,
            pltpu.SemaphoreType.DMA,
        ],
    )(x_bf, row_h, col_h)


def _mlp_half(gxi, gxj, ea_h, weights, ecount):
    W1a, W1b, W1c, W1d, b1r, W2b, b2r = weights
    return pl.pallas_call(
        _mlp_body,
        grid=(ecount // BE,),
        in_specs=[
            pl.BlockSpec((BE, D), lambda i: (i, 0)),
            pl.BlockSpec((BE, D), lambda i: (i, 0)),
            pl.BlockSpec((BE, DE), lambda i: (i, 0)),
            pl.BlockSpec((D, H), lambda i: (0, 0)),
            pl.BlockSpec((D, H), lambda i: (0, 0)),
            pl.BlockSpec((DE, H), lambda i: (0, 0)),
            pl.BlockSpec((D, H), lambda i: (0, 0)),
            pl.BlockSpec((1, H), lambda i: (0, 0)),
            pl.BlockSpec((H, D), lambda i: (0, 0)),
            pl.BlockSpec((1, D), lambda i: (0, 0)),
        ],
        out_specs=pl.BlockSpec((BE, D), lambda i: (i, 0)),
        out_shape=jax.ShapeDtypeStruct((ecount, D), jnp.float32),
    )(gxi, gxj, ea_h, W1a, W1b, W1c, W1d, b1r, W2b, b2r)


def _scatter_half(msg_h, col_h, init, ecount):
    return pl.kernel(
        functools.partial(_scatter_body, ecount),
        out_type=jax.ShapeDtypeStruct((NPAD, D), jnp.float32),
        mesh=_mesh(),
        compiler_params=_SC_SCATTER_PARAMS,
        cost_estimate=pl.CostEstimate(
            flops=ecount * D, transcendentals=0,
            bytes_accessed=ecount * (D * 4 + 4) * 33 // 32 + 2 * NPAD * D * 4),
        scratch_types=[
            pltpu.VMEM((SCH,), jnp.int32),
            pltpu.VMEM((SCH,), jnp.int32),
            pltpu.VMEM((SCH + 16,), jnp.int32),
            pltpu.VMEM((SCH + 16,), jnp.int32),
            pltpu.VMEM((SCH + 16,), jnp.int32),
            pltpu.VMEM((SCH + 16,), jnp.int32),
            pltpu.VMEM((GB, D), jnp.float32),
            pltpu.VMEM((GB, D), jnp.float32),
            pltpu.VMEM((NPW + 16, D), jnp.float32),
            pltpu.SMEM((2,), jnp.int32),
            pltpu.SemaphoreType.DMA,
            pltpu.SemaphoreType.DMA,
            pltpu.SemaphoreType.DMA,
            pltpu.SemaphoreType.DMA,
        ],
    )(msg_h, col_h, init)


def kernel(x, edge_index, edge_attr, W1, b1, W2, b2):
    row = edge_index[0]
    col = edge_index[1]
    bf = jnp.bfloat16
    x_bf = x.astype(bf)

    W1a = W1[0:D].astype(bf)
    W1b = W1[D:2 * D].astype(bf)
    W1c = W1[2 * D:2 * D + DE].astype(bf)
    W1d = W1[2 * D + DE:].astype(bf)
    weights = (W1a, W1b, W1c, W1d, b1.reshape(1, H), W2.astype(bf),
               b2.reshape(1, D))

    bounds = ((0, EH0), (EH0, EH1))
    scat = jnp.zeros((NPAD, D), jnp.float32)
    for start, ecount in bounds:
        row_h = lax.slice(row, (start,), (start + ecount,))
        col_h = lax.slice(col, (start,), (start + ecount,))
        ea_h = lax.slice(edge_attr, (start, 0), (start + ecount, DE))
        gxi, gxj = _gather_half(x_bf, row_h, col_h, ecount)
        msg_h = _mlp_half(gxi, gxj, ea_h, weights, ecount)
        scat = _scatter_half(msg_h, col_h, scat, ecount)

    BN = 1000
    out = pl.pallas_call(
        _tail,
        grid=(N // BN,),
        in_specs=[
            pl.BlockSpec((BN, D), lambda i: (i, 0)),
            pl.BlockSpec((BN, D), lambda i: (i, 0)),
        ],
        out_specs=pl.BlockSpec((BN, D), lambda i: (i, 0)),
        out_shape=jax.ShapeDtypeStruct((N, D), x.dtype),
    )(scat, x)
    return out


# speedup vs baseline: 1.3106x; 1.3106x over previous
import functools

import jax
import jax.numpy as jnp
from jax import lax
from jax.experimental import pallas as pl
from jax.experimental.pallas import tpu as pltpu
from jax.experimental.pallas import tpu_sc as plsc

N, E, D, DE, H = 10000, 160000, 256, 16, 512
NC, NS = 2, 16
NW = NC * NS
GC = 200
NPW = 320
NPAD = NW * NPW
SCH = 1600
UNR = 4
GB = 64
BE = 1600
EH0, EH1 = 83200, 76800


def _mesh():
    return plsc.VectorSubcoreMesh(
        core_axis_name="c", subcore_axis_name="s", num_cores=NC, num_subcores=NS
    )


_SC_GATHER_PARAMS = pltpu.CompilerParams(
    needs_layout_passes=False, use_tc_tiling_on_sc=False)
_SC_SCATTER_PARAMS = pltpu.CompilerParams(needs_layout_passes=False)


def _gather_body(ecount, x_hbm, row_hbm, col_hbm, gi_hbm, gj_hbm,
                 ridx, cidx, xi_v, xj_v, sem1, sem2):
    epw = ecount // NW
    wid = lax.axis_index("s") * NC + lax.axis_index("c")
    base = wid * epw

    def chunk(k, carry):
        off = base + k * GC
        pltpu.sync_copy(row_hbm.at[pl.ds(off, GC)], ridx)
        pltpu.sync_copy(col_hbm.at[pl.ds(off, GC)], cidx)
        cp1 = pltpu.async_copy(x_hbm.at[ridx], xi_v, sem1)
        cp2 = pltpu.async_copy(x_hbm.at[cidx], xj_v, sem2)
        cp1.wait()
        cp2.wait()
        pltpu.sync_copy(xi_v, gi_hbm.at[pl.ds(off, GC)])
        pltpu.sync_copy(xj_v, gj_hbm.at[pl.ds(off, GC)])
        return carry

    lax.fori_loop(0, epw // GC, chunk, 0)


def _bcast_last(v, lane15):
    dnums = lax.GatherDimensionNumbers(
        offset_dims=(), collapsed_slice_dims=(0,), start_index_map=(0,))
    return lax.gather(v, lane15.reshape(16, 1), dnums, (1,),
                      mode=lax.GatherScatterMode.PROMISE_IN_BOUNDS)


def _scatter_body(ecount, msg_hbm, col_hbm, init_hbm, scat_hbm,
                  cols0, cols1, mid0, mid1, mdst0, mdst1, rows0, rows1, out_v,
                  cnts, sg0, sg1, sc0, sc1):
    nch = ecount // SCH
    wid = lax.axis_index("s") * NC + lax.axis_index("c")
    lo = wid * NPW
    hi = lo + NPW
    pltpu.sync_copy(init_hbm.at[pl.ds(lo, NPW)], out_v.at[pl.ds(0, NPW)])
    zid = jnp.zeros((16,), jnp.int32)
    cols = (cols0, cols1)
    mid = (mid0, mid1)
    mdst = (mdst0, mdst1)
    rows = (rows0, rows1)
    sg = (sg0, sg1)
    sc = (sc0, sc1)

    def iz(j, carry):
        mid0[pl.ds(16 * j, 16)] = zid
        mid1[pl.ds(16 * j, 16)] = zid
        return carry

    lax.fori_loop(0, (SCH + 16) // 16, iz, 0)
    iota = lax.iota(jnp.int32, 16)
    hivec = jnp.zeros((16,), jnp.int32) + hi

    pltpu.async_copy(col_hbm.at[pl.ds(0, SCH)], cols[0], sc[0])

    def scan_issue(k, par):
        cbase = k * SCH
        pltpu.make_async_copy(col_hbm.at[pl.ds(cbase, SCH)], cols[par],
                              sc[par]).wait()

        @pl.when(k + 1 < nch)
        def _():
            pltpu.async_copy(col_hbm.at[pl.ds(cbase + SCH, SCH)],
                             cols[1 - par], sc[1 - par])

        cv0 = cols[par]
        midp = mid[par]
        mdstp = mdst[par]

        lane15 = jnp.zeros((16,), jnp.int32) + 15

        def scan(j, cntv):
            for u in range(UNR):
                o = 16 * (UNR * j + u)
                cv = cv0[pl.ds(o, 16)]
                m = (cv >= lo) & (cv < hi)
                cs = plsc.cumsum(jnp.where(m, 1, 0))
                idx = cntv + cs - 1
                plsc.store_scatter(midp, [idx], cbase + o + iota, mask=m)
                plsc.store_scatter(mdstp, [idx], cv, mask=m)
                cntv = cntv + _bcast_last(cs, lane15)
            return cntv

        cntv = lax.fori_loop(0, SCH // (16 * UNR), scan,
                             jnp.zeros((16,), jnp.int32))
        cnt = cntv[0]
        mdstp[pl.ds(cnt, 16)] = hivec
        cnts[par] = cnt
        nb = jnp.minimum((cnt + 15) // 16, GB // 16)
        for ng in (1, 2, 3, 4):

            @pl.when(nb == ng)
            def _(ng=ng):
                pltpu.async_copy(msg_hbm.at[midp.at[pl.ds(0, 16 * ng)]],
                                 rows[par].at[pl.ds(0, 16 * ng)], sg[par])

    def accum(rowsp, mdstp, base, nrows):
        def grp(g, carry):
            g16 = g * 16
            dstv = mdstp[pl.ds(base + g16, 16)]
            for t in range(16):
                r = dstv[t] - lo
                for v in range(16):
                    sl = pl.ds(16 * v, 16)
                    out_v[r, sl] = jnp.maximum(out_v[r, sl],
                                               rowsp[g16 + t, sl])
            return carry

        lax.fori_loop(0, (nrows + 15) // 16, grp, 0)

    def process(k, par):
        midp = mid[par]
        mdstp = mdst[par]
        rowsp = rows[par]
        cnt = cnts[par]
        nb = jnp.minimum((cnt + 15) // 16, GB // 16)
        for ng in (1, 2, 3, 4):

            @pl.when(nb == ng)
            def _(ng=ng):
                pltpu.make_async_copy(msg_hbm.at[midp.at[pl.ds(0, 16 * ng)]],
                                      rowsp.at[pl.ds(0, 16 * ng)],
                                      sg[par]).wait()

        accum(rowsp, mdstp, 0, jnp.minimum(cnt, GB))

        def bat(b, carry):
            bb = b * GB
            pltpu.async_copy(msg_hbm.at[midp.at[pl.ds(bb, GB)]], rowsp,
                             sg[par]).wait()
            accum(rowsp, mdstp, bb, jnp.minimum(cnt - bb, GB))
            return carry

        lax.fori_loop(1, (cnt + GB - 1) // GB, bat, 0)

    def pair(kk, carry):
        k0 = 2 * kk
        scan_issue(k0, 0)

        @pl.when(k0 > 0)
        def _():
            process(k0 - 1, 1)

        scan_issue(k0 + 1, 1)
        process(k0, 0)
        return carry

    lax.fori_loop(0, nch // 2, pair, 0)
    process(nch - 1, 1)
    pltpu.sync_copy(out_v.at[pl.ds(0, NPW)], scat_hbm.at[pl.ds(lo, NPW)])


def _mlp_body(xi_ref, xj_ref, ea_ref, w1a, w1b, w1c, w1d, b1_ref, w2, b2_ref,
              msg_ref):
    xi = xi_ref[...]
    xj = xj_ref[...]
    dirn = (2.3094 * jnp.tanh(xj.astype(jnp.float32) - xi.astype(jnp.float32))
            ).astype(jnp.bfloat16)
    ea = ea_ref[...].astype(jnp.bfloat16)

    def dot(a, b):
        return lax.dot_general(a, b, (((1,), (0,)), ((), ())),
                               preferred_element_type=jnp.float32)

    acc = dot(xi, w1a[...]) + dot(xj, w1b[...]) + dot(ea, w1c[...]) + dot(
        dirn, w1d[...])
    h = jnp.maximum(acc + b1_ref[...], 0.0).astype(jnp.bfloat16)
    msg_ref[...] = dot(h, w2[...]) + b2_ref[...]


def _tail(scat_ref, x_ref, o_ref):
    o_ref[...] = 4.0 * jnp.tanh(scat_ref[...] + 0.1 * x_ref[...])


def _gather_half(x_bf, row_h, col_h, ecount):
    return pl.kernel(
        functools.partial(_gather_body, ecount),
        out_type=[
            jax.ShapeDtypeStruct((ecount, D), jnp.bfloat16),
            jax.ShapeDtypeStruct((ecount, D), jnp.bfloat16),
        ],
        mesh=_mesh(),
        compiler_params=_SC_GATHER_PARAMS,
        cost_estimate=pl.CostEstimate(
            flops=0, transcendentals=0,
            bytes_accessed=ecount * (2 * D * 2 + 8) + N * D * 2),
        scratch_types=[
            pltpu.VMEM((GC,), jnp.int32),
            pltpu.VMEM((GC,), jnp.int32),
            pltpu.VMEM((GC, D), jnp.bfloat16),
            pltpu.VMEM((GC, D), jnp.bfloat16),
            pltpu.SemaphoreType.DMA,
            pltpu.SemaphoreType.DMA,
        ],
    )(x_bf, row_h, col_h)


def _mlp_half(gxi, gxj, ea_h, weights, ecount):
    W1a, W1b, W1c, W1d, b1r, W2b, b2r = weights
    return pl.pallas_call(
        _mlp_body,
        grid=(ecount // BE,),
        in_specs=[
            pl.BlockSpec((BE, D), lambda i: (i, 0)),
            pl.BlockSpec((BE, D), lambda i: (i, 0)),
            pl.BlockSpec((BE, DE), lambda i: (i, 0)),
            pl.BlockSpec((D, H), lambda i: (0, 0)),
            pl.BlockSpec((D, H), lambda i: (0, 0)),
            pl.BlockSpec((DE, H), lambda i: (0, 0)),
            pl.BlockSpec((D, H), lambda i: (0, 0)),
            pl.BlockSpec((1, H), lambda i: (0, 0)),
            pl.BlockSpec((H, D), lambda i: (0, 0)),
            pl.BlockSpec((1, D), lambda i: (0, 0)),
        ],
        out_specs=pl.BlockSpec((BE, D), lambda i: (i, 0)),
        out_shape=jax.ShapeDtypeStruct((ecount, D), jnp.float32),
    )(gxi, gxj, ea_h, W1a, W1b, W1c, W1d, b1r, W2b, b2r)


def _scatter_half(msg_h, col_h, init, ecount):
    return pl.kernel(
        functools.partial(_scatter_body, ecount),
        out_type=jax.ShapeDtypeStruct((NPAD, D), jnp.float32),
        mesh=_mesh(),
        compiler_params=_SC_SCATTER_PARAMS,
        cost_estimate=pl.CostEstimate(
            flops=ecount * D, transcendentals=0,
            bytes_accessed=ecount * (D * 4 + 4) * 33 // 32 + 2 * NPAD * D * 4),
        scratch_types=[
            pltpu.VMEM((SCH,), jnp.int32),
            pltpu.VMEM((SCH,), jnp.int32),
            pltpu.VMEM((SCH + 16,), jnp.int32),
            pltpu.VMEM((SCH + 16,), jnp.int32),
            pltpu.VMEM((SCH + 16,), jnp.int32),
            pltpu.VMEM((SCH + 16,), jnp.int32),
            pltpu.VMEM((GB, D), jnp.float32),
            pltpu.VMEM((GB, D), jnp.float32),
            pltpu.VMEM((NPW + 16, D), jnp.float32),
            pltpu.SMEM((2,), jnp.int32),
            pltpu.SemaphoreType.DMA,
            pltpu.SemaphoreType.DMA,
            pltpu.SemaphoreType.DMA,
            pltpu.SemaphoreType.DMA,
        ],
    )(msg_h, col_h, init)


def kernel(x, edge_index, edge_attr, W1, b1, W2, b2):
    row = edge_index[0]
    col = edge_index[1]
    bf = jnp.bfloat16
    x_bf = x.astype(bf)

    W1a = W1[0:D].astype(bf)
    W1b = W1[D:2 * D].astype(bf)
    W1c = W1[2 * D:2 * D + DE].astype(bf)
    W1d = W1[2 * D + DE:].astype(bf)
    weights = (W1a, W1b, W1c, W1d, b1.reshape(1, H), W2.astype(bf),
               b2.reshape(1, D))

    bounds = ((0, EH0), (EH0, EH1))
    scat = jnp.zeros((NPAD, D), jnp.float32)
    for start, ecount in bounds:
        row_h = lax.slice(row, (start,), (start + ecount,))
        col_h = lax.slice(col, (start,), (start + ecount,))
        ea_h = lax.slice(edge_attr, (start, 0), (start + ecount, DE))
        gxi, gxj = _gather_half(x_bf, row_h, col_h, ecount)
        msg_h = _mlp_half(gxi, gxj, ea_h, weights, ecount)
        scat = _scatter_half(msg_h, col_h, scat, ecount)

    BN = 1000
    out = pl.pallas_call(
        _tail,
        grid=(N // BN,),
        in_specs=[
            pl.BlockSpec((BN, D), lambda i: (i, 0)),
            pl.BlockSpec((BN, D), lambda i: (i, 0)),
        ],
        out_specs=pl.BlockSpec((BN, D), lambda i: (i, 0)),
        out_shape=jax.ShapeDtypeStruct((N, D), x.dtype),
    )(scat, x)
    return out
